# Initial kernel scaffold; baseline (speedup 1.0000x reference)
import jax
import jax.numpy as jnp
from jax.experimental import pallas as pl


def kernel(word_id_batch, tag_id_batch, deprel_id_batch, word_emb, tag_emb, deprel_emb, W1, b1, W2, b2):
    raise NotImplementedError("write your pallas kernel here")


# speedup vs baseline: 3.3978x; 3.3978x over previous
import functools

import jax
import jax.numpy as jnp
from jax import lax
from jax.experimental import pallas as pl
from jax.experimental.pallas import tpu as pltpu
from jax.experimental.pallas import tpu_sc as plsc

B = 16384
N_WORD_FEAT = 18
N_TAG_FEAT = 18
N_DEPREL_FEAT = 12
EMBED = 64
HIDDEN = 200
N_CLASSES = 80

NC = 2
NS = 16
NW = NC * NS

N_ROWS = B * N_WORD_FEAT
CHUNK = 128
ROWS_PER_W = N_ROWS // NW
CHUNKS_PER_W = ROWS_PER_W // CHUNK


def _sc_gather(table, idx2d):
    mesh = plsc.VectorSubcoreMesh(
        core_axis_name="c", subcore_axis_name="s", num_cores=NC, num_subcores=NS
    )

    @functools.partial(
        pl.kernel,
        out_type=jax.ShapeDtypeStruct((N_ROWS, EMBED), jnp.float32),
        mesh=mesh,
        scratch_types=[
            pltpu.VMEM((CHUNKS_PER_W, CHUNK), jnp.int32),
            pltpu.VMEM((CHUNK, EMBED), jnp.float32),
            pltpu.VMEM((CHUNK, EMBED), jnp.float32),
            pltpu.SemaphoreType.DMA,
            pltpu.SemaphoreType.DMA,
        ],
        compiler_params=pltpu.CompilerParams(use_tc_tiling_on_sc=False),
    )
    def gather_kernel(table_hbm, idx_hbm, out_hbm, idx_v, rows0, rows1, sem0, sem1):
        wid = lax.axis_index("s") * NC + lax.axis_index("c")
        chunk_base = wid * CHUNKS_PER_W
        pltpu.sync_copy(idx_hbm.at[pl.ds(chunk_base * 1, CHUNKS_PER_W)], idx_v)

        rows = (rows0, rows1)
        sems = (sem0, sem1)

        pltpu.async_copy(table_hbm.at[idx_v.at[0]], rows0, sem0)

        def body(j, _):
            @pl.when(j + 1 < CHUNKS_PER_W)
            def _fire():
                for par in range(2):

                    @pl.when(lax.rem(j + 1, 2) == par)
                    def _():
                        pltpu.async_copy(
                            table_hbm.at[idx_v.at[j + 1]], rows[par], sems[par]
                        )

            for par in range(2):

                @pl.when(lax.rem(j, 2) == par)
                def _():
                    pltpu.make_async_copy(
                        table_hbm.at[idx_v.at[j]], rows[par], sems[par]
                    ).wait()
                    pltpu.sync_copy(
                        rows[par],
                        out_hbm.at[pl.ds((chunk_base + j) * CHUNK, CHUNK)],
                    )

            return 0

        lax.fori_loop(0, CHUNKS_PER_W, body, 0)

    return gather_kernel(table, idx2d)


def _mlp_body(xw_ref, tag_ref, dep_ref, temb_ref, demb_ref, w1_ref, b1_ref,
              w2_ref, b2_ref, out_ref, pt_ref, pd_ref):
    blk = xw_ref.shape[0]

    @pl.when(pl.program_id(0) == 0)
    def _build_proj():
        for f in range(N_TAG_FEAT):
            base = N_WORD_FEAT * EMBED + f * EMBED
            pt_ref[f * EMBED:(f + 1) * EMBED, :] = jnp.dot(
                temb_ref[...], w1_ref[base:base + EMBED, :],
                preferred_element_type=jnp.float32)
        for f in range(N_DEPREL_FEAT):
            base = (N_WORD_FEAT + N_TAG_FEAT) * EMBED + f * EMBED
            pd_ref[f * EMBED:(f + 1) * EMBED, :] = jnp.dot(
                demb_ref[...], w1_ref[base:base + EMBED, :],
                preferred_element_type=jnp.float32)

    tag_ids = tag_ref[...]
    dep_ids = dep_ref[...]
    a_t = jnp.concatenate(
        [jnp.broadcast_to(tag_ids[:, f:f + 1], (blk, EMBED))
         for f in range(N_TAG_FEAT)], axis=1)
    a_d = jnp.concatenate(
        [jnp.broadcast_to(dep_ids[:, f:f + 1], (blk, EMBED))
         for f in range(N_DEPREL_FEAT)], axis=1)
    t_t = lax.rem(lax.broadcasted_iota(jnp.int32, (blk, N_TAG_FEAT * EMBED), 1),
                  EMBED)
    t_d = lax.rem(lax.broadcasted_iota(jnp.int32, (blk, N_DEPREL_FEAT * EMBED), 1),
                  EMBED)
    oh_t = (a_t == t_t).astype(jnp.float32)
    oh_d = (a_d == t_d).astype(jnp.float32)

    h = jnp.dot(xw_ref[...], w1_ref[0:N_WORD_FEAT * EMBED, :],
                preferred_element_type=jnp.float32)
    h = h + jnp.dot(oh_t, pt_ref[...], preferred_element_type=jnp.float32)
    h = h + jnp.dot(oh_d, pd_ref[...], preferred_element_type=jnp.float32)
    h = jnp.maximum(h + b1_ref[...], 0.0)
    out_ref[...] = jnp.dot(h, w2_ref[...],
                           preferred_element_type=jnp.float32) + b2_ref[...]


def _mlp(xw, tag_ids, dep_ids, tag_emb, deprel_emb, W1, b1, W2, b2):
    blk = 512
    grid = (B // blk,)
    return pl.pallas_call(
        _mlp_body,
        grid=grid,
        in_specs=[
            pl.BlockSpec((blk, N_WORD_FEAT * EMBED), lambda i: (i, 0)),
            pl.BlockSpec((blk, N_TAG_FEAT), lambda i: (i, 0)),
            pl.BlockSpec((blk, N_DEPREL_FEAT), lambda i: (i, 0)),
            pl.BlockSpec((EMBED, EMBED), lambda i: (0, 0)),
            pl.BlockSpec((EMBED, EMBED), lambda i: (0, 0)),
            pl.BlockSpec((W1.shape[0], HIDDEN), lambda i: (0, 0)),
            pl.BlockSpec((1, HIDDEN), lambda i: (0, 0)),
            pl.BlockSpec((HIDDEN, N_CLASSES), lambda i: (0, 0)),
            pl.BlockSpec((1, N_CLASSES), lambda i: (0, 0)),
        ],
        out_specs=pl.BlockSpec((blk, N_CLASSES), lambda i: (i, 0)),
        out_shape=jax.ShapeDtypeStruct((B, N_CLASSES), jnp.float32),
        scratch_shapes=[
            pltpu.VMEM((N_TAG_FEAT * EMBED, HIDDEN), jnp.float32),
            pltpu.VMEM((N_DEPREL_FEAT * EMBED, HIDDEN), jnp.float32),
        ],
    )(xw, tag_ids, dep_ids, tag_emb, deprel_emb, W1, b1, W2, b2)


def kernel(word_id_batch, tag_id_batch, deprel_id_batch, word_emb, tag_emb,
           deprel_emb, W1, b1, W2, b2):
    idx2d = word_id_batch.reshape(N_ROWS // CHUNK, CHUNK)
    gathered = _sc_gather(word_emb, idx2d)
    xw = gathered.reshape(B, N_WORD_FEAT * EMBED)
    return _mlp(xw, tag_id_batch, deprel_id_batch, tag_emb, deprel_emb,
                W1, b1.reshape(1, HIDDEN), W2, b2.reshape(1, N_CLASSES))


# speedup vs baseline: 3.7703x; 1.1096x over previous
import functools

import jax
import jax.numpy as jnp
from jax import lax
from jax.experimental import pallas as pl
from jax.experimental.pallas import tpu as pltpu
from jax.experimental.pallas import tpu_sc as plsc

B = 16384
N_WORD_FEAT = 18
N_TAG_FEAT = 18
N_DEPREL_FEAT = 12
EMBED = 64
HIDDEN = 200
N_CLASSES = 80

NC = 2
NS = 16
NW = NC * NS

NPAIR = N_WORD_FEAT // 2
N_WROWS = NPAIR * B
CHUNK = 128
WROWS_PER_W = N_WROWS // NW
CHUNKS_PER_W = WROWS_PER_W // CHUNK


def _sc_gather(table, idx_e2d, idx_o2d):
    mesh = plsc.VectorSubcoreMesh(
        core_axis_name="c", subcore_axis_name="s", num_cores=NC, num_subcores=NS
    )

    @functools.partial(
        pl.kernel,
        out_type=jax.ShapeDtypeStruct((N_WROWS, 2 * EMBED), jnp.float32),
        mesh=mesh,
        scratch_types=[
            pltpu.VMEM((CHUNKS_PER_W, CHUNK), jnp.int32),
            pltpu.VMEM((CHUNKS_PER_W, CHUNK), jnp.int32),
            pltpu.VMEM((CHUNK, EMBED), jnp.float32),
            pltpu.VMEM((CHUNK, EMBED), jnp.float32),
            pltpu.VMEM((CHUNK, EMBED), jnp.float32),
            pltpu.VMEM((CHUNK, EMBED), jnp.float32),
            pltpu.SemaphoreType.DMA,
            pltpu.SemaphoreType.DMA,
        ],
        compiler_params=pltpu.CompilerParams(use_tc_tiling_on_sc=False),
    )
    def gather_kernel(table_hbm, idxe_hbm, idxo_hbm, out_hbm,
                      idxe_v, idxo_v, rowse0, rowso0, rowse1, rowso1,
                      sem0, sem1):
        wid = lax.axis_index("s") * NC + lax.axis_index("c")
        chunk_base = wid * CHUNKS_PER_W
        pltpu.sync_copy(idxe_hbm.at[pl.ds(chunk_base, CHUNKS_PER_W)], idxe_v)
        pltpu.sync_copy(idxo_hbm.at[pl.ds(chunk_base, CHUNKS_PER_W)], idxo_v)

        rows_e = (rowse0, rowse1)
        rows_o = (rowso0, rowso1)
        sems = (sem0, sem1)

        def fire(j, par):
            pltpu.async_copy(table_hbm.at[idxe_v.at[j]], rows_e[par], sems[par])
            pltpu.async_copy(table_hbm.at[idxo_v.at[j]], rows_o[par], sems[par])

        def drain(j, par):
            pltpu.make_async_copy(
                table_hbm.at[idxe_v.at[j]], rows_e[par], sems[par]).wait()
            pltpu.make_async_copy(
                table_hbm.at[idxo_v.at[j]], rows_o[par], sems[par]).wait()
            base = (chunk_base + j) * CHUNK
            pltpu.sync_copy(
                rows_e[par], out_hbm.at[pl.ds(base, CHUNK), pl.ds(0, EMBED)])
            pltpu.sync_copy(
                rows_o[par], out_hbm.at[pl.ds(base, CHUNK), pl.ds(EMBED, EMBED)])

        fire(0, 0)

        def body(j, _):
            @pl.when(j + 1 < CHUNKS_PER_W)
            def _fire():
                for par in range(2):
                    @pl.when(lax.rem(j + 1, 2) == par)
                    def _():
                        fire(j + 1, par)

            for par in range(2):
                @pl.when(lax.rem(j, 2) == par)
                def _():
                    drain(j, par)

            return 0

        lax.fori_loop(0, CHUNKS_PER_W, body, 0)

    return gather_kernel(table, idx_e2d, idx_o2d)


def _mlp_body(g2_ref, tag_ref, dep_ref, temb_ref, demb_ref, w1_ref, b1_ref,
              w2_ref, b2_ref, out_ref, pt_ref, pd_ref):
    blk = tag_ref.shape[0]

    @pl.when(pl.program_id(0) == 0)
    def _build_proj():
        for f in range(N_TAG_FEAT):
            base = N_WORD_FEAT * EMBED + f * EMBED
            pt_ref[f * EMBED:(f + 1) * EMBED, :] = jnp.dot(
                temb_ref[...], w1_ref[base:base + EMBED, :],
                preferred_element_type=jnp.float32)
        for f in range(N_DEPREL_FEAT):
            base = (N_WORD_FEAT + N_TAG_FEAT) * EMBED + f * EMBED
            pd_ref[f * EMBED:(f + 1) * EMBED, :] = jnp.dot(
                demb_ref[...], w1_ref[base:base + EMBED, :],
                preferred_element_type=jnp.float32)

    h = jnp.dot(g2_ref[0], w1_ref[0:2 * EMBED, :],
                preferred_element_type=jnp.float32)
    for j in range(1, NPAIR):
        h = h + jnp.dot(g2_ref[j], w1_ref[j * 2 * EMBED:(j + 1) * 2 * EMBED, :],
                        preferred_element_type=jnp.float32)

    tag_ids = tag_ref[...]
    dep_ids = dep_ref[...]
    a_t = jnp.concatenate(
        [jnp.broadcast_to(tag_ids[:, f:f + 1], (blk, EMBED))
         for f in range(N_TAG_FEAT)], axis=1)
    a_d = jnp.concatenate(
        [jnp.broadcast_to(dep_ids[:, f:f + 1], (blk, EMBED))
         for f in range(N_DEPREL_FEAT)], axis=1)
    t_t = lax.rem(lax.broadcasted_iota(jnp.int32, (blk, N_TAG_FEAT * EMBED), 1),
                  EMBED)
    t_d = lax.rem(lax.broadcasted_iota(jnp.int32, (blk, N_DEPREL_FEAT * EMBED), 1),
                  EMBED)
    oh_t = (a_t == t_t).astype(jnp.float32)
    oh_d = (a_d == t_d).astype(jnp.float32)

    h = h + jnp.dot(oh_t, pt_ref[...], preferred_element_type=jnp.float32)
    h = h + jnp.dot(oh_d, pd_ref[...], preferred_element_type=jnp.float32)
    h = jnp.maximum(h + b1_ref[...], 0.0)
    out_ref[...] = jnp.dot(h, w2_ref[...],
                           preferred_element_type=jnp.float32) + b2_ref[...]


def _mlp(g2, tag_ids, dep_ids, tag_emb, deprel_emb, W1, b1, W2, b2):
    blk = 512
    grid = (B // blk,)
    return pl.pallas_call(
        _mlp_body,
        grid=grid,
        in_specs=[
            pl.BlockSpec((NPAIR, blk, 2 * EMBED), lambda i: (0, i, 0)),
            pl.BlockSpec((blk, N_TAG_FEAT), lambda i: (i, 0)),
            pl.BlockSpec((blk, N_DEPREL_FEAT), lambda i: (i, 0)),
            pl.BlockSpec((EMBED, EMBED), lambda i: (0, 0)),
            pl.BlockSpec((EMBED, EMBED), lambda i: (0, 0)),
            pl.BlockSpec((W1.shape[0], HIDDEN), lambda i: (0, 0)),
            pl.BlockSpec((1, HIDDEN), lambda i: (0, 0)),
            pl.BlockSpec((HIDDEN, N_CLASSES), lambda i: (0, 0)),
            pl.BlockSpec((1, N_CLASSES), lambda i: (0, 0)),
        ],
        out_specs=pl.BlockSpec((blk, N_CLASSES), lambda i: (i, 0)),
        out_shape=jax.ShapeDtypeStruct((B, N_CLASSES), jnp.float32),
        scratch_shapes=[
            pltpu.VMEM((N_TAG_FEAT * EMBED, HIDDEN), jnp.float32),
            pltpu.VMEM((N_DEPREL_FEAT * EMBED, HIDDEN), jnp.float32),
        ],
    )(g2, tag_ids, dep_ids, tag_emb, deprel_emb, W1, b1, W2, b2)


def kernel(word_id_batch, tag_id_batch, deprel_id_batch, word_emb, tag_emb,
           deprel_emb, W1, b1, W2, b2):
    wi3 = word_id_batch.reshape(B, NPAIR, 2)
    idx_e = wi3[:, :, 0].T.reshape(N_WROWS // CHUNK, CHUNK)
    idx_o = wi3[:, :, 1].T.reshape(N_WROWS // CHUNK, CHUNK)
    gathered = _sc_gather(word_emb, idx_e, idx_o)
    g2 = gathered.reshape(NPAIR, B, 2 * EMBED)
    return _mlp(g2, tag_id_batch, deprel_id_batch, tag_emb, deprel_emb,
                W1, b1.reshape(1, HIDDEN), W2, b2.reshape(1, N_CLASSES))
